# baseline (speedup 1.0000x reference)
import jax
import jax.numpy as jnp
from jax.experimental import pallas as pl
from jax.experimental.pallas import tpu as pltpu

_B = 1024
_K = 1024
_D = 64
_BB = 64


def _som_kernel(x_ref, lmt_ref, qd_ref, out_ref):
    x = x_ref[...]
    lmt = lmt_ref[...]
    xlm = jax.lax.dot_general(
        x, lmt, (((1,), (0,)), ((), ())), preferred_element_type=jnp.float32
    )
    x2 = jnp.sum(x * x, axis=1, keepdims=True)
    lm2 = jnp.sum(lmt * lmt, axis=0, keepdims=True)
    dist = x2 + lm2 - 2.0 * xlm
    dmin = jnp.min(dist, axis=1, keepdims=True)
    iota = jax.lax.broadcasted_iota(jnp.int32, dist.shape, 1)
    idx = jnp.min(jnp.where(dist == dmin, iota, _K), axis=1, keepdims=True)
    onehot = (iota == idx).astype(jnp.float32)
    h = jax.lax.dot_general(
        onehot, qd_ref[...], (((1,), (0,)), ((), ())),
        preferred_element_type=jnp.float32,
    )
    out_ref[...] = h[:, None, :] * (x[:, :, None] - lmt[None, :, :])


@jax.jit
def kernel(x, landmarks, qd):
    grid = (_B // _BB,)
    out_t = pl.pallas_call(
        _som_kernel,
        grid=grid,
        in_specs=[
            pl.BlockSpec((_BB, _D), lambda i: (i, 0)),
            pl.BlockSpec((_D, _K), lambda i: (0, 0)),
            pl.BlockSpec((_K, _K), lambda i: (0, 0)),
        ],
        out_specs=pl.BlockSpec((_BB, _D, _K), lambda i: (i, 0, 0)),
        out_shape=jax.ShapeDtypeStruct((_B, _D, _K), jnp.float32),
        compiler_params=pltpu.CompilerParams(
            dimension_semantics=("parallel",),
        ),
    )(x, landmarks.T, qd)
    return jnp.swapaxes(out_t, 1, 2)
